# Initial kernel scaffold; baseline (speedup 1.0000x reference)
import jax
import jax.numpy as jnp
from jax.experimental import pallas as pl


def kernel(pred, trg, biased_trg, biased_offset):
    raise NotImplementedError("write your pallas kernel here")


# speedup vs baseline: 5.7670x; 5.7670x over previous
import functools
import math

import jax
import jax.numpy as jnp
import numpy as np
from jax.experimental import pallas as pl
from jax.experimental.pallas import tpu as pltpu

_B, _S, _V = 4, 512, 32000
_N = _B * _S
_LS = 0.1
_PAD = 0
_TRG_FACTOR = 1.0 - _LS

_U = float(np.float32(_LS / (_V - 2)))
_C1 = float(np.float32(_U) * np.float32(np.log(np.float32(_U))))

_BR = 256
_BV = 6400


def _xlogx_cols(d):
    safe = jnp.where(d > 0, d, 1.0)
    return d * jnp.log(safe)


def _body(trg_ref, bt_ref, off_ref, pred_ref, out_ref):
    j = pl.program_id(1)
    t = trg_ref[...]
    bt = bt_ref[...]
    off = off_ref[...]

    a = _TRG_FACTOR * (1.0 - off)
    o = off * _TRG_FACTOR

    d_b = _U + o
    d_t = a + jnp.where(bt == t, o, 0.0)
    d_0 = jnp.where(bt == _PAD, o, 0.0)
    g_b = _xlogx_cols(d_b)
    g_t = _xlogx_cols(d_t)
    g_0 = _xlogx_cols(d_0)

    p = pred_ref[...]
    col = jax.lax.broadcasted_iota(jnp.int32, p.shape, 1) + j * _BV

    out = _C1 - _U * p
    out = jnp.where(col == bt, g_b - d_b * p, out)
    out = jnp.where(col == t, g_t - d_t * p, out)
    out = jnp.where(col == _PAD, g_0 - d_0 * p, out)
    out = jnp.where(t == _PAD, 0.0, out)
    out_ref[...] = out


@jax.jit
def kernel(pred, trg, biased_trg, biased_offset):
    pred2 = pred.reshape(_N, _V)
    t2 = trg.reshape(_N, 1)
    bt2 = biased_trg.reshape(_N, 1)
    off2 = biased_offset.reshape(_N, 1)

    grid = (_N // _BR, _V // _BV)
    row_spec = pl.BlockSpec((_BR, 1), lambda i, j: (i, 0))
    return pl.pallas_call(
        _body,
        grid=grid,
        in_specs=[
            row_spec,
            row_spec,
            row_spec,
            pl.BlockSpec((_BR, _BV), lambda i, j: (i, j)),
        ],
        out_specs=pl.BlockSpec((_BR, _BV), lambda i, j: (i, j)),
        out_shape=jax.ShapeDtypeStruct((_N, _V), jnp.float32),
        compiler_params=pltpu.CompilerParams(
            dimension_semantics=("parallel", "parallel"),
        ),
    )(t2, bt2, off2, pred2)
